# baseline (speedup 1.0000x reference)
import functools

import jax
import jax.numpy as jnp
from jax import lax
from jax.experimental import pallas as pl
from jax.experimental.pallas import tpu as pltpu
from jax.experimental.pallas import tpu_sc as plsc

_B, _C1, _C2, _H, _W = 8, 384, 192, 224, 224
_HW = _H * _W
_WH, _WW = 32, 32
_PB = 7168
_NSC = 4
_NWIN = (_H // _WH) * (_W // _WW)


def _pool_body(x_ref, s_ref):
    first = (pl.program_id(1) == 0) & (pl.program_id(2) == 0)
    acc = x_ref[0, 0, pl.ds(0, 8), :]
    for wt in range(_WW // 8):
        for h in range(_WH):
            if wt == 0 and h == 0:
                continue
            acc = acc + x_ref[0, h, pl.ds(wt * 8, 8), :]
    t = acc[0:4, :] + acc[4:8, :]
    t = t[0:2, :] + t[2:4, :]
    s = t[0:1, :] + t[1:2, :]

    @pl.when(first)
    def _():
        s_ref[0, :, :] = s

    @pl.when(jnp.logical_not(first))
    def _():
        s_ref[0, :, :] = s_ref[0, :, :] + s


def _channel_sums_tc(xt):
    nb = _B - _NSC
    return pl.pallas_call(
        _pool_body,
        grid=(nb, _H // _WH, _W // _WW),
        in_specs=[
            pl.BlockSpec((1, _WH, _WW, _C1), lambda b, i, j: (b + _NSC, i, j, 0)),
        ],
        out_specs=pl.BlockSpec((1, 1, _C1), lambda b, i, j: (b, 0, 0)),
        out_shape=jax.ShapeDtypeStruct((nb, 1, _C1), jnp.float32),
    )(xt)


def _sc_pool_kernel(x_hbm, out_hbm, buf, buf2, accs, row, sem, sem2):
    wid = lax.axis_index("s") * 2 + lax.axis_index("c")
    nwin = _NSC * _NWIN

    for g in range((nwin + 31) // 32):
        w = g * 32 + wid

        @pl.when(w < nwin)
        def _():
            b = w // _NWIN
            wi = w % _NWIN
            i = wi // (_W // _WW)
            j = wi % (_W // _WW)

            def zbody(t, _):
                accs[pl.ds(t * 16, 16)] = jnp.zeros((16,), jnp.float32)
                return 0

            lax.fori_loop(0, 8 * _C1 // 16, zbody, 0)

            pieces = [(wt, half) for wt in range(_WW // 8) for half in range(2)]
            bufs = (buf, buf2)
            sems = (sem, sem2)

            def _issue(p):
                wt, half = pieces[p]
                return pltpu.async_copy(
                    x_hbm.at[
                        b,
                        pl.ds(i * _WH + half * 16, 16),
                        pl.ds(j * _WW + wt * 8, 8),
                        :,
                    ],
                    bufs[p % 2],
                    sems[p % 2],
                )

            cps = {0: _issue(0)}
            for p in range(8):
                cps[p].wait()
                if p < 7:
                    cps[p + 1] = _issue(p + 1)
                cur = bufs[p % 2]

                def pbody(t, _, cur=cur):
                    s = t // (_C1 // 16)
                    k = t % (_C1 // 16)
                    aoff = s * _C1 + k * 16
                    acc = accs[pl.ds(aoff, 16)]
                    coff = k * 16
                    for h in range(16):
                        acc = acc + cur[h, s, pl.ds(coff, 16)]
                    accs[pl.ds(aoff, 16)] = acc
                    return 0

                lax.fori_loop(0, 8 * _C1 // 16, pbody, 0)

            def tbody(k, _):
                o = k * 16
                a0 = accs[pl.ds(0 * _C1 + o, 16)]
                a1 = accs[pl.ds(1 * _C1 + o, 16)]
                a2 = accs[pl.ds(2 * _C1 + o, 16)]
                a3 = accs[pl.ds(3 * _C1 + o, 16)]
                a4 = accs[pl.ds(4 * _C1 + o, 16)]
                a5 = accs[pl.ds(5 * _C1 + o, 16)]
                a6 = accs[pl.ds(6 * _C1 + o, 16)]
                a7 = accs[pl.ds(7 * _C1 + o, 16)]
                row[pl.ds(o, 16)] = ((a0 + a4) + (a2 + a6)) + ((a1 + a5) + (a3 + a7))
                return 0

            lax.fori_loop(0, _C1 // 16, tbody, 0)
            pltpu.sync_copy(row, out_hbm.at[b, wi, :])


def _channel_sums_sc(xt5):
    scmesh = plsc.VectorSubcoreMesh(core_axis_name="c", subcore_axis_name="s")
    k = pl.kernel(
        _sc_pool_kernel,
        mesh=scmesh,
        out_type=jax.ShapeDtypeStruct((_NSC, _NWIN, _C1), jnp.float32),
        scratch_types=[
            pltpu.VMEM((16, 8, _C1), jnp.float32),
            pltpu.VMEM((16, 8, _C1), jnp.float32),
            pltpu.VMEM((8 * _C1,), jnp.float32),
            pltpu.VMEM((_C1,), jnp.float32),
            pltpu.SemaphoreType.DMA,
            pltpu.SemaphoreType.DMA,
        ],
    )
    parts = k(xt5)
    acc = parts[:, 0, :]
    for wi in range(1, _NWIN):
        acc = acc + parts[:, wi, :]
    return acc


def _gather_body(ys_ref, x_ref, o_ref, oh_ref):
    @pl.when(pl.program_id(1) == 0)
    def _():
        v = ys_ref[0, 0, :]
        a = v[None, :]
        bv = v[:, None]
        jio = lax.broadcasted_iota(jnp.int32, (_C1, _C1), 1)
        iio = lax.broadcasted_iota(jnp.int32, (_C1, _C1), 0)
        before = (a > bv) | ((a == bv) & (jio < iio))
        rank = jnp.sum(before.astype(jnp.int32), axis=1)
        pio = lax.broadcasted_iota(jnp.int32, (_C1, _C2), 1)
        oh_ref[...] = (rank[:, None] == pio).astype(jnp.float32)

    o_ref[0, :, :] = jax.lax.dot_general(
        oh_ref[...], x_ref[0, :, :],
        dimension_numbers=(((0,), (1,)), ((), ())),
        precision=jax.lax.Precision.DEFAULT,
        preferred_element_type=jnp.float32,
    )


def _gather_channels(xt2, ys):
    return pl.pallas_call(
        _gather_body,
        grid=(_B, _HW // _PB),
        in_specs=[
            pl.BlockSpec((1, 1, _C1), lambda b, p: (b, 0, 0)),
            pl.BlockSpec((1, _PB, _C1), lambda b, p: (b, p, 0)),
        ],
        out_specs=pl.BlockSpec((1, _C2, _PB), lambda b, p: (b, 0, p)),
        out_shape=jax.ShapeDtypeStruct((_B, _C2, _HW), jnp.float32),
        scratch_shapes=[pltpu.VMEM((_C1, _C2), jnp.float32)],
    )(ys.reshape(_B, 1, _C1), xt2)


def kernel(x, conv_w):
    xt = jnp.transpose(x, (0, 2, 3, 1))
    sums_sc = _channel_sums_sc(xt)
    sums_tc = _channel_sums_tc(xt).reshape(_B - _NSC, _C1)
    sums = jnp.concatenate([sums_sc, sums_tc], axis=0)
    y = sums / jnp.float32(_HW)
    yp = jnp.pad(y, ((0, 0), (1, 1)))
    yc = conv_w[0] * yp[:, :-2] + conv_w[1] * yp[:, 1:-1] + conv_w[2] * yp[:, 2:]
    ys = jax.nn.sigmoid(yc)
    out = _gather_channels(xt.reshape(_B, _HW, _C1), ys)
    return out.reshape(_B, _C2, _H, _W)


# speedup vs baseline: 1.0432x; 1.0432x over previous
import functools

import jax
import jax.numpy as jnp
from jax import lax
from jax.experimental import pallas as pl
from jax.experimental.pallas import tpu as pltpu
from jax.experimental.pallas import tpu_sc as plsc

_B, _C1, _C2, _H, _W = 8, 384, 192, 224, 224
_HW = _H * _W
_WH, _WW = 32, 32
_PB = 7168
_NWIN = (_H // _WH) * (_W // _WW)
_TWIN = _B * _NWIN
_SCW = 172
_NSCB = (_SCW + _NWIN - 1) // _NWIN


def _pool_body(x_ref, s_ref):
    acc = x_ref[0, 0, pl.ds(0, 8), :]
    for wt in range(_WW // 8):
        for h in range(_WH):
            if wt == 0 and h == 0:
                continue
            acc = acc + x_ref[0, h, pl.ds(wt * 8, 8), :]
    t = acc[0:4, :] + acc[4:8, :]
    t = t[0:2, :] + t[2:4, :]
    s_ref[0, :, :] = t[0:1, :] + t[1:2, :]


def _channel_sums_tc(xt):
    nw = _TWIN - _SCW

    def imap(w):
        g = w + _SCW
        wi = g % _NWIN
        return (g // _NWIN, wi // (_W // _WW), wi % (_W // _WW), 0)

    return pl.pallas_call(
        _pool_body,
        grid=(nw,),
        in_specs=[pl.BlockSpec((1, _WH, _WW, _C1), imap)],
        out_specs=pl.BlockSpec((1, 1, _C1), lambda w: (w, 0, 0)),
        out_shape=jax.ShapeDtypeStruct((nw, 1, _C1), jnp.float32),
    )(xt)


def _sc_pool_kernel(x_hbm, out_hbm, buf, buf2, accs, row, sem, sem2):
    wid = lax.axis_index("s") * 2 + lax.axis_index("c")
    nwin = _SCW

    for g in range((nwin + 31) // 32):
        w = g * 32 + wid

        @pl.when(w < nwin)
        def _():
            b = w // _NWIN
            wi = w % _NWIN
            i = wi // (_W // _WW)
            j = wi % (_W // _WW)

            def zbody(t, _):
                accs[pl.ds(t * 16, 16)] = jnp.zeros((16,), jnp.float32)
                return 0

            lax.fori_loop(0, 8 * _C1 // 16, zbody, 0)

            pieces = [(wt, half) for wt in range(_WW // 8) for half in range(2)]
            bufs = (buf, buf2)
            sems = (sem, sem2)

            def _issue(p):
                wt, half = pieces[p]
                return pltpu.async_copy(
                    x_hbm.at[
                        b,
                        pl.ds(i * _WH + half * 16, 16),
                        pl.ds(j * _WW + wt * 8, 8),
                        :,
                    ],
                    bufs[p % 2],
                    sems[p % 2],
                )

            cps = {0: _issue(0)}
            for p in range(8):
                cps[p].wait()
                if p < 7:
                    cps[p + 1] = _issue(p + 1)
                cur = bufs[p % 2]

                def pbody(t, _, cur=cur):
                    s = t // (_C1 // 16)
                    k = t % (_C1 // 16)
                    aoff = s * _C1 + k * 16
                    acc = accs[pl.ds(aoff, 16)]
                    coff = k * 16
                    for h in range(16):
                        acc = acc + cur[h, s, pl.ds(coff, 16)]
                    accs[pl.ds(aoff, 16)] = acc
                    return 0

                lax.fori_loop(0, 8 * _C1 // 16, pbody, 0)

            def tbody(k, _):
                o = k * 16
                a0 = accs[pl.ds(0 * _C1 + o, 16)]
                a1 = accs[pl.ds(1 * _C1 + o, 16)]
                a2 = accs[pl.ds(2 * _C1 + o, 16)]
                a3 = accs[pl.ds(3 * _C1 + o, 16)]
                a4 = accs[pl.ds(4 * _C1 + o, 16)]
                a5 = accs[pl.ds(5 * _C1 + o, 16)]
                a6 = accs[pl.ds(6 * _C1 + o, 16)]
                a7 = accs[pl.ds(7 * _C1 + o, 16)]
                row[pl.ds(o, 16)] = ((a0 + a4) + (a2 + a6)) + ((a1 + a5) + (a3 + a7))
                return 0

            lax.fori_loop(0, _C1 // 16, tbody, 0)
            pltpu.sync_copy(row, out_hbm.at[b, wi, :])


def _channel_sums_sc(xt5):
    scmesh = plsc.VectorSubcoreMesh(core_axis_name="c", subcore_axis_name="s")
    k = pl.kernel(
        _sc_pool_kernel,
        mesh=scmesh,
        out_type=jax.ShapeDtypeStruct((_NSCB, _NWIN, _C1), jnp.float32),
        scratch_types=[
            pltpu.VMEM((16, 8, _C1), jnp.float32),
            pltpu.VMEM((16, 8, _C1), jnp.float32),
            pltpu.VMEM((8 * _C1,), jnp.float32),
            pltpu.VMEM((_C1,), jnp.float32),
            pltpu.SemaphoreType.DMA,
            pltpu.SemaphoreType.DMA,
        ],
    )
    return k(xt5)


def _gather_body(ys_ref, x_ref, o_ref, oh_ref):
    @pl.when(pl.program_id(1) == 0)
    def _():
        v = ys_ref[0, 0, :]
        a = v[None, :]
        bv = v[:, None]
        jio = lax.broadcasted_iota(jnp.int32, (_C1, _C1), 1)
        iio = lax.broadcasted_iota(jnp.int32, (_C1, _C1), 0)
        before = (a > bv) | ((a == bv) & (jio < iio))
        rank = jnp.sum(before.astype(jnp.int32), axis=1)
        pio = lax.broadcasted_iota(jnp.int32, (_C1, _C2), 1)
        oh_ref[...] = (rank[:, None] == pio).astype(jnp.float32)

    o_ref[0, :, :] = jax.lax.dot_general(
        oh_ref[...], x_ref[0, :, :],
        dimension_numbers=(((0,), (1,)), ((), ())),
        precision=jax.lax.Precision.DEFAULT,
        preferred_element_type=jnp.float32,
    )


def _gather_channels(xt2, ys):
    return pl.pallas_call(
        _gather_body,
        grid=(_B, _HW // _PB),
        in_specs=[
            pl.BlockSpec((1, 1, _C1), lambda b, p: (b, 0, 0)),
            pl.BlockSpec((1, _PB, _C1), lambda b, p: (b, p, 0)),
        ],
        out_specs=pl.BlockSpec((1, _C2, _PB), lambda b, p: (b, 0, p)),
        out_shape=jax.ShapeDtypeStruct((_B, _C2, _HW), jnp.float32),
        scratch_shapes=[pltpu.VMEM((_C1, _C2), jnp.float32)],
    )(ys.reshape(_B, 1, _C1), xt2)


def kernel(x, conv_w):
    xt = jnp.transpose(x, (0, 2, 3, 1))
    scp = _channel_sums_sc(xt)
    tcp = _channel_sums_tc(xt)
    rows = []
    for b in range(_B):
        acc = None
        for wi in range(_NWIN):
            g = b * _NWIN + wi
            term = scp[b, wi, :] if g < _SCW else tcp[g - _SCW, 0, :]
            acc = term if acc is None else acc + term
        rows.append(acc)
    sums = jnp.stack(rows, axis=0)
    y = sums / jnp.float32(_HW)
    yp = jnp.pad(y, ((0, 0), (1, 1)))
    yc = conv_w[0] * yp[:, :-2] + conv_w[1] * yp[:, 1:-1] + conv_w[2] * yp[:, 2:]
    ys = jax.nn.sigmoid(yc)
    out = _gather_channels(xt.reshape(_B, _HW, _C1), ys)
    return out.reshape(_B, _C2, _H, _W)


# speedup vs baseline: 1.0481x; 1.0047x over previous
import functools

import jax
import jax.numpy as jnp
from jax import lax
from jax.experimental import pallas as pl
from jax.experimental.pallas import tpu as pltpu
from jax.experimental.pallas import tpu_sc as plsc

_B, _C1, _C2, _H, _W = 8, 384, 192, 224, 224
_HW = _H * _W
_WH, _WW = 32, 32
_PB = 7168
_NSC = 3
_NWIN = (_H // _WH) * (_W // _WW)


def _pool_body(x_ref, s_ref):
    first = (pl.program_id(1) == 0) & (pl.program_id(2) == 0)
    acc = x_ref[0, 0, pl.ds(0, 8), :]
    for wt in range(_WW // 8):
        for h in range(_WH):
            if wt == 0 and h == 0:
                continue
            acc = acc + x_ref[0, h, pl.ds(wt * 8, 8), :]
    t = acc[0:4, :] + acc[4:8, :]
    t = t[0:2, :] + t[2:4, :]
    s = t[0:1, :] + t[1:2, :]

    @pl.when(first)
    def _():
        s_ref[0, :, :] = s

    @pl.when(jnp.logical_not(first))
    def _():
        s_ref[0, :, :] = s_ref[0, :, :] + s


def _channel_sums_tc(xt):
    nb = _B - _NSC
    return pl.pallas_call(
        _pool_body,
        grid=(nb, _H // _WH, _W // _WW),
        in_specs=[
            pl.BlockSpec((1, _WH, _WW, _C1), lambda b, i, j: (b + _NSC, i, j, 0)),
        ],
        out_specs=pl.BlockSpec((1, 1, _C1), lambda b, i, j: (b, 0, 0)),
        out_shape=jax.ShapeDtypeStruct((nb, 1, _C1), jnp.float32),
    )(xt)


def _sc_pool_kernel(x_hbm, out_hbm, buf, buf2, accs, row, sem, sem2):
    wid = lax.axis_index("s") * 2 + lax.axis_index("c")
    nwin = _NSC * _NWIN

    for g in range((nwin + 31) // 32):
        w = g * 32 + wid

        @pl.when(w < nwin)
        def _():
            b = w // _NWIN
            wi = w % _NWIN
            i = wi // (_W // _WW)
            j = wi % (_W // _WW)

            def zbody(t, _):
                accs[pl.ds(t * 16, 16)] = jnp.zeros((16,), jnp.float32)
                return 0

            lax.fori_loop(0, 8 * _C1 // 16, zbody, 0)

            pieces = [(wt, half) for wt in range(_WW // 8) for half in range(2)]
            bufs = (buf, buf2)
            sems = (sem, sem2)

            def _issue(p):
                wt, half = pieces[p]
                return pltpu.async_copy(
                    x_hbm.at[
                        b,
                        pl.ds(i * _WH + half * 16, 16),
                        pl.ds(j * _WW + wt * 8, 8),
                        :,
                    ],
                    bufs[p % 2],
                    sems[p % 2],
                )

            cps = {0: _issue(0)}
            for p in range(8):
                cps[p].wait()
                if p < 7:
                    cps[p + 1] = _issue(p + 1)
                cur = bufs[p % 2]

                def pbody(t, _, cur=cur):
                    s = t // (_C1 // 16)
                    k = t % (_C1 // 16)
                    aoff = s * _C1 + k * 16
                    acc = accs[pl.ds(aoff, 16)]
                    coff = k * 16
                    for h in range(16):
                        acc = acc + cur[h, s, pl.ds(coff, 16)]
                    accs[pl.ds(aoff, 16)] = acc
                    return 0

                lax.fori_loop(0, 8 * _C1 // 16, pbody, 0)

            def tbody(k, _):
                o = k * 16
                a0 = accs[pl.ds(0 * _C1 + o, 16)]
                a1 = accs[pl.ds(1 * _C1 + o, 16)]
                a2 = accs[pl.ds(2 * _C1 + o, 16)]
                a3 = accs[pl.ds(3 * _C1 + o, 16)]
                a4 = accs[pl.ds(4 * _C1 + o, 16)]
                a5 = accs[pl.ds(5 * _C1 + o, 16)]
                a6 = accs[pl.ds(6 * _C1 + o, 16)]
                a7 = accs[pl.ds(7 * _C1 + o, 16)]
                row[pl.ds(o, 16)] = ((a0 + a4) + (a2 + a6)) + ((a1 + a5) + (a3 + a7))
                return 0

            lax.fori_loop(0, _C1 // 16, tbody, 0)
            pltpu.sync_copy(row, out_hbm.at[b, wi, :])


def _channel_sums_sc(xt5):
    scmesh = plsc.VectorSubcoreMesh(core_axis_name="c", subcore_axis_name="s")
    k = pl.kernel(
        _sc_pool_kernel,
        mesh=scmesh,
        out_type=jax.ShapeDtypeStruct((_NSC, _NWIN, _C1), jnp.float32),
        scratch_types=[
            pltpu.VMEM((16, 8, _C1), jnp.float32),
            pltpu.VMEM((16, 8, _C1), jnp.float32),
            pltpu.VMEM((8 * _C1,), jnp.float32),
            pltpu.VMEM((_C1,), jnp.float32),
            pltpu.SemaphoreType.DMA,
            pltpu.SemaphoreType.DMA,
        ],
    )
    parts = k(xt5)
    acc = parts[:, 0, :]
    for wi in range(1, _NWIN):
        acc = acc + parts[:, wi, :]
    return acc


def _gather_body(ys_ref, x_ref, o_ref, oh_ref):
    @pl.when(pl.program_id(1) == 0)
    def _():
        v = ys_ref[0, 0, :]
        a = v[None, :]
        bv = v[:, None]
        jio = lax.broadcasted_iota(jnp.int32, (_C1, _C1), 1)
        iio = lax.broadcasted_iota(jnp.int32, (_C1, _C1), 0)
        before = (a > bv) | ((a == bv) & (jio < iio))
        rank = jnp.sum(before.astype(jnp.int32), axis=1)
        pio = lax.broadcasted_iota(jnp.int32, (_C1, _C2), 1)
        oh_ref[...] = (rank[:, None] == pio).astype(jnp.float32)

    o_ref[0, :, :] = jax.lax.dot_general(
        oh_ref[...], x_ref[0, :, :],
        dimension_numbers=(((0,), (1,)), ((), ())),
        precision=jax.lax.Precision.DEFAULT,
        preferred_element_type=jnp.float32,
    )


def _gather_channels(xt2, ys):
    return pl.pallas_call(
        _gather_body,
        grid=(_B, _HW // _PB),
        in_specs=[
            pl.BlockSpec((1, 1, _C1), lambda b, p: (b, 0, 0)),
            pl.BlockSpec((1, _PB, _C1), lambda b, p: (b, p, 0)),
        ],
        out_specs=pl.BlockSpec((1, _C2, _PB), lambda b, p: (b, 0, p)),
        out_shape=jax.ShapeDtypeStruct((_B, _C2, _HW), jnp.float32),
        scratch_shapes=[pltpu.VMEM((_C1, _C2), jnp.float32)],
    )(ys.reshape(_B, 1, _C1), xt2)


def kernel(x, conv_w):
    xt = jnp.transpose(x, (0, 2, 3, 1))
    sums_sc = _channel_sums_sc(xt)
    sums_tc = _channel_sums_tc(xt).reshape(_B - _NSC, _C1)
    sums = jnp.concatenate([sums_sc, sums_tc], axis=0)
    y = sums / jnp.float32(_HW)
    yp = jnp.pad(y, ((0, 0), (1, 1)))
    yc = conv_w[0] * yp[:, :-2] + conv_w[1] * yp[:, 1:-1] + conv_w[2] * yp[:, 2:]
    ys = jax.nn.sigmoid(yc)
    out = _gather_channels(xt.reshape(_B, _HW, _C1), ys)
    return out.reshape(_B, _C2, _H, _W)
